# baseline (speedup 1.0000x reference)
import functools

import jax
import jax.numpy as jnp
from jax import lax
from jax.experimental import pallas as pl
from jax.experimental.pallas import tpu as pltpu
from jax.experimental.pallas import tpu_sc as plsc

N_NODES = 10000
D = 256
DH = D // 2
VOCAB_PAD = 408
E_TOTAL = 160000
E_PAD = 163840
N_TILES = 16
NR = 632
NR_LAST = N_NODES - 15 * NR
MAGIC = 26547
CA = 32
R_ROUTE = 512
CAP = 544
CM = 32

_GATHER_DNUMS = lax.GatherDimensionNumbers(
    offset_dims=(), collapsed_slice_dims=(0,), start_index_map=(0,))


def _lane_shuffle(v, idx):
    return lax.gather(v, idx[:, None], _GATHER_DNUMS, (1,),
                      mode=lax.GatherScatterMode.PROMISE_IN_BOUNDS)


def _mm_bias(x, w, b, block_rows):
    R, d = x.shape
    a = w.shape[0]

    def body(x_ref, w_ref, b_ref, o_ref):
        o_ref[...] = lax.dot_general(
            x_ref[...], w_ref[...], (((1,), (1,)), ((), ())),
            preferred_element_type=jnp.float32) + b_ref[...]

    return pl.pallas_call(
        body,
        grid=(R // block_rows,),
        in_specs=[pl.BlockSpec((block_rows, d), lambda i: (i, 0)),
                  pl.BlockSpec((d, a), lambda i: (0, 0)),
                  pl.BlockSpec((1, a), lambda i: (0, 0))],
        out_specs=pl.BlockSpec((block_rows, a), lambda i: (i, 0)),
        out_shape=jax.ShapeDtypeStruct((R, a), jnp.float32),
    )(x, w, b)


def _mm_final(p0a, p1a, p0b, p1b, w, block_rows):
    R = p0a.shape[0]
    a = w.shape[0]

    def body(a0, a1, b0, b1, w_ref, o_ref):
        x = jnp.concatenate([a0[...] + a1[...], b0[...] + b1[...]], axis=1)
        o_ref[...] = lax.dot_general(
            x, w_ref[...], (((1,), (1,)), ((), ())),
            preferred_element_type=jnp.float32)

    half = pl.BlockSpec((block_rows, DH), lambda i: (i, 0))
    return pl.pallas_call(
        body,
        grid=(R // block_rows,),
        in_specs=[half, half, half, half,
                  pl.BlockSpec((D, a), lambda i: (0, 0))],
        out_specs=pl.BlockSpec((block_rows, a), lambda i: (i, 0)),
        out_shape=jax.ShapeDtypeStruct((R, a), jnp.float32),
    )(p0a, p1a, p0b, p1b, w)


def _sc_alpha(hid_attn, rel_attn, r2, q_rel, wal_ext, sub, rel, ridx):
    mesh = plsc.VectorSubcoreMesh(core_axis_name="c", subcore_axis_name="s")
    e_per_w = E_PAD // (2 * N_TILES)
    n_chunks = e_per_w // CA

    @functools.partial(
        pl.kernel,
        mesh=mesh,
        out_type=jax.ShapeDtypeStruct((E_PAD,), jnp.float32),
        scratch_types=[
            pltpu.VMEM((D + 16,), jnp.float32),
            pltpu.VMEM((CA,), jnp.int32),
            pltpu.VMEM((CA,), jnp.int32),
            pltpu.VMEM((CA,), jnp.int32),
            pltpu.VMEM((CA,), jnp.int32),
            pltpu.VMEM((CA, D), jnp.float32),
            pltpu.VMEM((CA, D), jnp.float32),
            pltpu.VMEM((CA, D), jnp.float32),
            pltpu.VMEM((CA,), jnp.float32),
            pltpu.SemaphoreType.DMA,
            pltpu.SemaphoreType.DMA,
            pltpu.SemaphoreType.DMA,
        ],
    )
    def body(ha_hbm, ra_hbm, r2_hbm, qrel_hbm, wal_hbm, sub_hbm, rel_hbm,
             ridx_hbm, alpha_hbm, wal_v, sub_v, rel_v, ridx_v, qidx_v,
             ha_v, ra_v, cq_v, al_v, sem1, sem2, sem3):
        c = lax.axis_index("c")
        s = lax.axis_index("s")
        w = s * 2 + c
        pltpu.sync_copy(wal_hbm, wal_v)
        lanes = lax.iota(jnp.int32, 16)
        wal_b = wal_v[pl.ds(D, 16)]
        wal_regs = [wal_v[pl.ds(j * 16, 16)] for j in range(D // 16)]

        def chunk_body(t, carry):
            base = w * e_per_w + t * CA
            pltpu.sync_copy(sub_hbm.at[pl.ds(base, CA)], sub_v)
            pltpu.sync_copy(rel_hbm.at[pl.ds(base, CA)], rel_v)
            pltpu.sync_copy(ridx_hbm.at[pl.ds(base, CA)], ridx_v)
            cp1 = pltpu.async_copy(ha_hbm.at[sub_v], ha_v, sem1)
            cp2 = pltpu.async_copy(ra_hbm.at[rel_v], ra_v, sem2)
            cp3 = pltpu.async_copy(qrel_hbm.at[ridx_v], qidx_v, sem3)
            cp1.wait()
            cp3.wait()
            cp4 = pltpu.async_copy(r2_hbm.at[qidx_v], cq_v, sem3)
            cp2.wait()
            cp4.wait()

            def group_body(g, carry2):
                avec = jnp.zeros((16,), jnp.float32)
                for e in range(16):
                    acc = jnp.zeros((16,), jnp.float32)
                    ei = g * 16 + e
                    for j in range(D // 16):
                        x = (ha_v[ei, pl.ds(j * 16, 16)]
                             + ra_v[ei, pl.ds(j * 16, 16)]
                             + cq_v[ei, pl.ds(j * 16, 16)])
                        acc = acc + jnp.maximum(x, 0.0) * wal_regs[j]
                    for sh in (8, 4, 2, 1):
                        acc = acc + _lane_shuffle(acc, lanes ^ sh)
                    av = 1.0 / (1.0 + jnp.exp(-(acc + wal_b)))
                    avec = jnp.where(lanes == e, av, avec)
                al_v[pl.ds(g * 16, 16)] = avec
                return carry2

            lax.fori_loop(0, CA // 16, group_body, 0)
            pltpu.sync_copy(al_v, alpha_hbm.at[pl.ds(base, CA)])
            return carry

        lax.fori_loop(0, n_chunks, chunk_body, 0)

    return body(hid_attn, rel_attn, r2, q_rel, wal_ext, sub, rel, ridx)


def _sc_accumulate(hid_h, rel_h, alpha, sub, rel, obj):
    mesh = plsc.VectorSubcoreMesh(core_axis_name="c", subcore_axis_name="s")
    e_per_w = E_PAD // (2 * N_TILES)
    n_rounds = e_per_w // R_ROUTE

    @functools.partial(
        pl.kernel,
        mesh=mesh,
        out_type=(jax.ShapeDtypeStruct((N_NODES, DH), jnp.float32),
                  jax.ShapeDtypeStruct((N_NODES, DH), jnp.float32)),
        scratch_types=[
            pltpu.VMEM((NR, DH), jnp.float32),
            pltpu.VMEM((N_TILES * CAP,), jnp.int32),
            pltpu.VMEM((CAP,), jnp.int32),
            pltpu.VMEM((R_ROUTE,), jnp.int32),
            pltpu.VMEM((CM,), jnp.int32),
            pltpu.VMEM((CM,), jnp.int32),
            pltpu.VMEM((CM,), jnp.int32),
            pltpu.VMEM((CM,), jnp.int32),
            pltpu.VMEM((CM,), jnp.float32),
            pltpu.VMEM((CM, DH), jnp.float32),
            pltpu.VMEM((CM, DH), jnp.float32),
            pltpu.VMEM_SHARED((N_TILES * N_TILES * CAP,), jnp.int32),
            pltpu.SMEM((N_TILES,), jnp.int32),
            pltpu.SemaphoreType.DMA,
            pltpu.SemaphoreType.DMA,
            pltpu.SemaphoreType.DMA,
            pltpu.SemaphoreType.DMA,
        ],
    )
    def body(hid_hbm, rel_hbm, al_hbm, sub_hbm, rele_hbm, obj_hbm,
             out0_hbm, out1_hbm, acc, lst, work, objs, eidc, subc, relc,
             objc, alc, hs_v, hr_v, exch, cnt, sem1, sem2, sem3, sem4):
        c = lax.axis_index("c")
        s = lax.axis_index("s")
        lanes = lax.iota(jnp.int32, 16)
        zv = jnp.zeros((16,), jnp.float32)
        node_base = s * NR

        def zero_acc(r, carry):
            for j in range(DH // 16):
                acc[r, pl.ds(j * 16, 16)] = zv
            return carry

        lax.fori_loop(0, NR, zero_acc, 0)

        def round_body(r, carry):
            ebase = c * (E_PAD // 2) + s * e_per_w + r * R_ROUTE
            pltpu.sync_copy(obj_hbm.at[pl.ds(ebase, R_ROUTE)], objs)
            m1 = jnp.full((16,), -1, jnp.int32)

            def clear_body(k, carry2):
                lst[pl.ds(k * 16, 16)] = m1
                return carry2

            lax.fori_loop(0, N_TILES * CAP // 16, clear_body, 0)
            for d in range(N_TILES):
                cnt[d] = 0

            def route_body(g, carry2):
                ov = objs[pl.ds(g * 16, 16)]
                for e in range(16):
                    o = ov[e]
                    eid = ebase + g * 16 + e

                    @pl.when(o >= 0)
                    def _(o=o, eid=eid):
                        d = (o * MAGIC) >> 24
                        cn = cnt[d]
                        lst[pl.ds(d * CAP + cn, 16)] = jnp.full(
                            (16,), eid, jnp.int32)
                        cnt[d] = cn + 1
                return carry2

            lax.fori_loop(0, R_ROUTE // 16, route_body, 0)
            for d in range(N_TILES):
                lst[pl.ds(d * CAP + cnt[d], 16)] = m1

            pltpu.sync_copy(lst, exch.at[pl.ds(s * (N_TILES * CAP),
                                               N_TILES * CAP)])
            plsc.subcore_barrier()

            def drain_body(src, carry2_outer):
                pltpu.sync_copy(
                    exch.at[pl.ds(src * (N_TILES * CAP) + s * CAP, CAP)],
                    work)

                def chunk_body(k, carry2):
                    vs = [work[pl.ds(k * CM + q * 16, 16)]
                          for q in range(CM // 16)]
                    mx = vs[0]
                    for v in vs[1:]:
                        mx = jnp.maximum(mx, v)
                    for sh in (8, 4, 2, 1):
                        mx = jnp.maximum(mx, _lane_shuffle(mx, lanes ^ sh))

                    @pl.when(mx[0] >= 0)
                    def _(vs=vs, k=k):
                        for q in range(CM // 16):
                            eidc[pl.ds(q * 16, 16)] = jnp.maximum(vs[q], 0)
                        g1 = pltpu.async_copy(sub_hbm.at[eidc], subc, sem1)
                        g2 = pltpu.async_copy(rele_hbm.at[eidc], relc, sem2)
                        g3 = pltpu.async_copy(obj_hbm.at[eidc], objc, sem3)
                        g4 = pltpu.async_copy(al_hbm.at[eidc], alc, sem4)
                        g1.wait()
                        g2.wait()
                        g3.wait()
                        g4.wait()
                        g5 = pltpu.async_copy(hid_hbm.at[subc], hs_v, sem1)
                        g6 = pltpu.async_copy(rel_hbm.at[relc], hr_v, sem2)
                        g5.wait()
                        g6.wait()
                        for q in range(CM // 16):
                            rawv = vs[q]
                            ov = objc[pl.ds(q * 16, 16)]
                            av = alc[pl.ds(q * 16, 16)]
                            for e in range(16):
                                ei = q * 16 + e

                                @pl.when(rawv[e] >= 0)
                                def _(ei=ei, row=ov[e] - node_base,
                                      asc=av[e]):
                                    afull = jnp.full((16,), asc, jnp.float32)
                                    for j in range(DH // 16):
                                        m = (hs_v[ei, pl.ds(j * 16, 16)]
                                             + hr_v[ei, pl.ds(j * 16, 16)])
                                        plsc.addupdate(
                                            acc.at[row, pl.ds(j * 16, 16)],
                                            afull * m)

                    return carry2

                lax.fori_loop(0, CAP // CM, chunk_body, 0)
                return carry2_outer

            lax.fori_loop(0, N_TILES, drain_body, 0)
            plsc.subcore_barrier()
            return carry

        lax.fori_loop(0, n_rounds, round_body, 0)

        def write_to(out_hbm):
            pl.when(s < N_TILES - 1)(lambda: pltpu.sync_copy(
                acc.at[pl.ds(0, NR)], out_hbm.at[pl.ds(node_base, NR)]))
            pl.when(s == N_TILES - 1)(lambda: pltpu.sync_copy(
                acc.at[pl.ds(0, NR_LAST)],
                out_hbm.at[pl.ds(node_base, NR_LAST)]))

        pl.when(c == 0)(lambda: write_to(out0_hbm))
        pl.when(c == 1)(lambda: write_to(out1_hbm))

    return body(hid_h, rel_h, alpha, sub, rel, obj)


def kernel(q_sub, q_rel, hidden, edges, n_node, old_nodes_new_idx, rela_embed,
           Ws, Wr, Wqr_w, Wqr_b, walpha_w, walpha_b, W_h):
    pad = E_PAD - E_TOTAL
    sub = jnp.pad(edges[:, 4], (0, pad))
    rel = jnp.pad(edges[:, 2], (0, pad))
    obj = jnp.pad(edges[:, 5], (0, pad), constant_values=-1)
    ridx = jnp.pad(edges[:, 0], (0, pad))
    rela_pad = jnp.pad(rela_embed, ((0, VOCAB_PAD - rela_embed.shape[0]),
                                    (0, 0)))
    zb = jnp.zeros((1, D), jnp.float32)
    hid_attn = _mm_bias(hidden, Ws, zb, 1000)
    rel_attn = _mm_bias(rela_pad, Wr, zb, VOCAB_PAD)
    r2 = _mm_bias(rela_pad, Wqr_w, Wqr_b.reshape(1, -1), VOCAB_PAD)
    wal_ext = jnp.concatenate(
        [walpha_w[0], jnp.full((16,), walpha_b[0], jnp.float32)])

    alpha = _sc_alpha(hid_attn, rel_attn, r2, q_rel, wal_ext, sub, rel, ridx)

    p0a, p1a = _sc_accumulate(hidden[:, :DH], rela_pad[:, :DH],
                              alpha, sub, rel, obj)
    p0b, p1b = _sc_accumulate(hidden[:, DH:], rela_pad[:, DH:],
                              alpha, sub, rel, obj)
    return _mm_final(p0a, p1a, p0b, p1b, W_h, 1000)


# speedup vs baseline: 1.6729x; 1.6729x over previous
import functools

import jax
import jax.numpy as jnp
from jax import lax
from jax.experimental import pallas as pl
from jax.experimental.pallas import tpu as pltpu
from jax.experimental.pallas import tpu_sc as plsc

N_NODES = 10000
D = 256
DH = D // 2
VOCAB_PAD = 408
E_TOTAL = 160000
E_PAD = 163840
N_TILES = 16
NR = 632
NR_LAST = N_NODES - 15 * NR
MAGIC = 26547
CA = 32
R_ROUTE = 512
CAP = 576
CM = 64
EID_BITS = 18
EID_MASK = (1 << EID_BITS) - 1

_GATHER_DNUMS = lax.GatherDimensionNumbers(
    offset_dims=(), collapsed_slice_dims=(0,), start_index_map=(0,))


def _lane_shuffle(v, idx):
    return lax.gather(v, idx[:, None], _GATHER_DNUMS, (1,),
                      mode=lax.GatherScatterMode.PROMISE_IN_BOUNDS)


def _mm_bias(x, w, b, block_rows):
    R, d = x.shape
    a = w.shape[0]

    def body(x_ref, w_ref, b_ref, o_ref):
        o_ref[...] = lax.dot_general(
            x_ref[...], w_ref[...], (((1,), (1,)), ((), ())),
            preferred_element_type=jnp.float32) + b_ref[...]

    return pl.pallas_call(
        body,
        grid=(R // block_rows,),
        in_specs=[pl.BlockSpec((block_rows, d), lambda i: (i, 0)),
                  pl.BlockSpec((d, a), lambda i: (0, 0)),
                  pl.BlockSpec((1, a), lambda i: (0, 0))],
        out_specs=pl.BlockSpec((block_rows, a), lambda i: (i, 0)),
        out_shape=jax.ShapeDtypeStruct((R, a), jnp.float32),
    )(x, w, b)


def _mm_final(p0a, p1a, p0b, p1b, w, block_rows):
    R = p0a.shape[0]
    a = w.shape[0]

    def body(a0, a1, b0, b1, w_ref, o_ref):
        x = jnp.concatenate([a0[...] + a1[...], b0[...] + b1[...]], axis=1)
        o_ref[...] = lax.dot_general(
            x, w_ref[...], (((1,), (1,)), ((), ())),
            preferred_element_type=jnp.float32)

    half = pl.BlockSpec((block_rows, DH), lambda i: (i, 0))
    return pl.pallas_call(
        body,
        grid=(R // block_rows,),
        in_specs=[half, half, half, half,
                  pl.BlockSpec((D, a), lambda i: (0, 0))],
        out_specs=pl.BlockSpec((block_rows, a), lambda i: (i, 0)),
        out_shape=jax.ShapeDtypeStruct((R, a), jnp.float32),
    )(p0a, p1a, p0b, p1b, w)


def _sc_alpha(hid_attn, rel_attn, r2, q_rel, wal_ext, sub, rel, ridx):
    mesh = plsc.VectorSubcoreMesh(core_axis_name="c", subcore_axis_name="s")
    e_per_w = E_PAD // (2 * N_TILES)
    n_chunks = e_per_w // CA

    @functools.partial(
        pl.kernel,
        mesh=mesh,
        out_type=jax.ShapeDtypeStruct((E_PAD,), jnp.float32),
        scratch_types=[
            pltpu.VMEM((D + 16,), jnp.float32),
            pltpu.VMEM((CA,), jnp.int32),
            pltpu.VMEM((CA,), jnp.int32),
            pltpu.VMEM((CA,), jnp.int32),
            pltpu.VMEM((CA,), jnp.int32),
            pltpu.VMEM((CA, D), jnp.float32),
            pltpu.VMEM((CA, D), jnp.float32),
            pltpu.VMEM((CA, D), jnp.float32),
            pltpu.VMEM((CA,), jnp.float32),
            pltpu.SemaphoreType.DMA,
            pltpu.SemaphoreType.DMA,
            pltpu.SemaphoreType.DMA,
        ],
    )
    def body(ha_hbm, ra_hbm, r2_hbm, qrel_hbm, wal_hbm, sub_hbm, rel_hbm,
             ridx_hbm, alpha_hbm, wal_v, sub_v, rel_v, ridx_v, qidx_v,
             ha_v, ra_v, cq_v, al_v, sem1, sem2, sem3):
        c = lax.axis_index("c")
        s = lax.axis_index("s")
        w = s * 2 + c
        pltpu.sync_copy(wal_hbm, wal_v)
        lanes = lax.iota(jnp.int32, 16)
        wal_b = wal_v[pl.ds(D, 16)]
        wal_regs = [wal_v[pl.ds(j * 16, 16)] for j in range(D // 16)]

        def chunk_body(t, carry):
            base = w * e_per_w + t * CA
            pltpu.sync_copy(sub_hbm.at[pl.ds(base, CA)], sub_v)
            pltpu.sync_copy(rel_hbm.at[pl.ds(base, CA)], rel_v)
            pltpu.sync_copy(ridx_hbm.at[pl.ds(base, CA)], ridx_v)
            cp1 = pltpu.async_copy(ha_hbm.at[sub_v], ha_v, sem1)
            cp2 = pltpu.async_copy(ra_hbm.at[rel_v], ra_v, sem2)
            cp3 = pltpu.async_copy(qrel_hbm.at[ridx_v], qidx_v, sem3)
            cp1.wait()
            cp3.wait()
            cp4 = pltpu.async_copy(r2_hbm.at[qidx_v], cq_v, sem3)
            cp2.wait()
            cp4.wait()

            def group_body(g, carry2):
                avec = jnp.zeros((16,), jnp.float32)
                for e in range(16):
                    acc = jnp.zeros((16,), jnp.float32)
                    ei = g * 16 + e
                    for j in range(D // 16):
                        x = (ha_v[ei, pl.ds(j * 16, 16)]
                             + ra_v[ei, pl.ds(j * 16, 16)]
                             + cq_v[ei, pl.ds(j * 16, 16)])
                        acc = acc + jnp.maximum(x, 0.0) * wal_regs[j]
                    for sh in (8, 4, 2, 1):
                        acc = acc + _lane_shuffle(acc, lanes ^ sh)
                    av = 1.0 / (1.0 + jnp.exp(-(acc + wal_b)))
                    avec = jnp.where(lanes == e, av, avec)
                al_v[pl.ds(g * 16, 16)] = avec
                return carry2

            lax.fori_loop(0, CA // 16, group_body, 0)
            pltpu.sync_copy(al_v, alpha_hbm.at[pl.ds(base, CA)])
            return carry

        lax.fori_loop(0, n_chunks, chunk_body, 0)

    return body(hid_attn, rel_attn, r2, q_rel, wal_ext, sub, rel, ridx)


def _sc_accumulate(hid_h, rel_h, alpha, sub, rel, obj):
    mesh = plsc.VectorSubcoreMesh(core_axis_name="c", subcore_axis_name="s")
    e_per_w = E_PAD // (2 * N_TILES)
    n_rounds = e_per_w // R_ROUTE

    @functools.partial(
        pl.kernel,
        mesh=mesh,
        out_type=(jax.ShapeDtypeStruct((N_NODES, DH), jnp.float32),
                  jax.ShapeDtypeStruct((N_NODES, DH), jnp.float32),
                  jax.ShapeDtypeStruct((2 * N_TILES * 16,), jnp.int32)),
        scratch_types=[
            pltpu.VMEM((NR, DH), jnp.float32),
            pltpu.VMEM((N_TILES * CAP,), jnp.int32),
            pltpu.VMEM((CAP,), jnp.int32),
            pltpu.VMEM((R_ROUTE,), jnp.int32),
            pltpu.VMEM((CM,), jnp.int32),
            pltpu.VMEM((CM,), jnp.int32),
            pltpu.VMEM((CM,), jnp.int32),
            pltpu.VMEM((CM,), jnp.float32),
            pltpu.VMEM((32,), jnp.int32),
            pltpu.VMEM((16,), jnp.int32),
            pltpu.VMEM((CM, DH), jnp.float32),
            pltpu.VMEM((CM, DH), jnp.float32),
            pltpu.VMEM_SHARED((N_TILES * N_TILES * CAP,), jnp.int32),
            pltpu.SMEM((N_TILES,), jnp.int32),
            pltpu.SemaphoreType.DMA,
            pltpu.SemaphoreType.DMA,
            pltpu.SemaphoreType.DMA,
        ],
    )
    def body(hid_hbm, rel_hbm, al_hbm, sub_hbm, rele_hbm, obj_hbm,
             out0_hbm, out1_hbm, cnts_hbm, acc, lst, work, objs, eidc,
             subc, relc, alc, cntv, csrc, hs_v, hr_v, exch, cnt,
             sem1, sem2, sem3):
        c = lax.axis_index("c")
        s = lax.axis_index("s")
        lanes = lax.iota(jnp.int32, 16)
        zv = jnp.zeros((16,), jnp.float32)
        node_base = s * NR
        cidx = c * (N_TILES * 16) + lanes * 16 + s

        def zero_acc(r, carry):
            for j in range(DH // 16):
                acc[r, pl.ds(j * 16, 16)] = zv
            return carry

        lax.fori_loop(0, NR, zero_acc, 0)

        def round_body(r, carry):
            ebase = c * (E_PAD // 2) + s * e_per_w + r * R_ROUTE
            pltpu.sync_copy(obj_hbm.at[pl.ds(ebase, R_ROUTE)], objs)
            for d in range(N_TILES):
                cnt[d] = 0

            def route_body(g, carry2):
                ov = objs[pl.ds(g * 16, 16)]
                for e in range(16):
                    o = ov[e]
                    eid = ebase + g * 16 + e

                    @pl.when(o >= 0)
                    def _(o=o, eid=eid):
                        d = (o * MAGIC) >> 24
                        packed = eid | ((o - d * NR) << EID_BITS)
                        cn = cnt[d]
                        lst[pl.ds(d * CAP + cn, 16)] = jnp.full(
                            (16,), packed, jnp.int32)
                        cnt[d] = cn + 1
                return carry2

            lax.fori_loop(0, R_ROUTE // 16, route_body, 0)

            for d in range(N_TILES):
                cntv[pl.ds(d, 16)] = jnp.full((16,), cnt[d], jnp.int32)
            pltpu.sync_copy(cntv.at[pl.ds(0, 16)],
                            cnts_hbm.at[pl.ds((c * N_TILES + s) * 16, 16)])
            pltpu.sync_copy(lst, exch.at[pl.ds(s * (N_TILES * CAP),
                                               N_TILES * CAP)])
            plsc.subcore_barrier()
            pltpu.async_copy(cnts_hbm.at[cidx], csrc, sem1).wait()
            cv = csrc[pl.ds(0, 16)]

            def drain_body(src, carry2_outer):
                nv = _lane_shuffle(cv, (lanes + src) & 15)
                n_src = nv[0]

                def do_src():
                    pltpu.sync_copy(
                        exch.at[pl.ds(src * (N_TILES * CAP) + s * CAP,
                                      CAP)], work)
                    n_chunks = (n_src + CM - 1) >> 6

                    def chunk_body(k, carry2):
                        kbase = k * CM
                        vs = [work[pl.ds(kbase + q * 16, 16)]
                              for q in range(CM // 16)]
                        for q in range(CM // 16):
                            eidc[pl.ds(q * 16, 16)] = jnp.minimum(
                                vs[q] & EID_MASK, E_PAD - 1)
                        g1 = pltpu.async_copy(sub_hbm.at[eidc], subc, sem1)
                        g2 = pltpu.async_copy(rele_hbm.at[eidc], relc, sem2)
                        g4 = pltpu.async_copy(al_hbm.at[eidc], alc, sem3)
                        g1.wait()
                        g2.wait()
                        g4.wait()
                        g5 = pltpu.async_copy(hid_hbm.at[subc], hs_v, sem1)
                        g6 = pltpu.async_copy(rel_hbm.at[relc], hr_v, sem2)
                        g5.wait()
                        g6.wait()
                        for q in range(CM // 16):
                            rawv = vs[q]
                            av = alc[pl.ds(q * 16, 16)]
                            for e in range(16):
                                ei = q * 16 + e

                                @pl.when(kbase + ei < n_src)
                                def _(ei=ei,
                                      row=rawv[e] >> EID_BITS,
                                      asc=av[e]):
                                    afull = jnp.full((16,), asc,
                                                     jnp.float32)
                                    for j in range(DH // 16):
                                        m = (hs_v[ei, pl.ds(j * 16, 16)]
                                             + hr_v[ei, pl.ds(j * 16, 16)])
                                        plsc.addupdate(
                                            acc.at[row, pl.ds(j * 16, 16)],
                                            afull * m)

                        return carry2

                    lax.fori_loop(0, n_chunks, chunk_body, 0)

                pl.when(n_src > 0)(do_src)
                return carry2_outer

            lax.fori_loop(0, N_TILES, drain_body, 0)
            plsc.subcore_barrier()
            return carry

        lax.fori_loop(0, n_rounds, round_body, 0)

        def write_to(out_hbm):
            pl.when(s < N_TILES - 1)(lambda: pltpu.sync_copy(
                acc.at[pl.ds(0, NR)], out_hbm.at[pl.ds(node_base, NR)]))
            pl.when(s == N_TILES - 1)(lambda: pltpu.sync_copy(
                acc.at[pl.ds(0, NR_LAST)],
                out_hbm.at[pl.ds(node_base, NR_LAST)]))

        pl.when(c == 0)(lambda: write_to(out0_hbm))
        pl.when(c == 1)(lambda: write_to(out1_hbm))

    return body(hid_h, rel_h, alpha, sub, rel, obj)


def kernel(q_sub, q_rel, hidden, edges, n_node, old_nodes_new_idx, rela_embed,
           Ws, Wr, Wqr_w, Wqr_b, walpha_w, walpha_b, W_h):
    pad = E_PAD - E_TOTAL
    sub = jnp.pad(edges[:, 4], (0, pad))
    rel = jnp.pad(edges[:, 2], (0, pad))
    obj = jnp.pad(edges[:, 5], (0, pad), constant_values=-1)
    ridx = jnp.pad(edges[:, 0], (0, pad))
    rela_pad = jnp.pad(rela_embed, ((0, VOCAB_PAD - rela_embed.shape[0]),
                                    (0, 0)))
    zb = jnp.zeros((1, D), jnp.float32)
    hid_attn = _mm_bias(hidden, Ws, zb, 1000)
    rel_attn = _mm_bias(rela_pad, Wr, zb, VOCAB_PAD)
    r2 = _mm_bias(rela_pad, Wqr_w, Wqr_b.reshape(1, -1), VOCAB_PAD)
    wal_ext = jnp.concatenate(
        [walpha_w[0], jnp.full((16,), walpha_b[0], jnp.float32)])

    alpha = _sc_alpha(hid_attn, rel_attn, r2, q_rel, wal_ext, sub, rel, ridx)

    p0a, p1a, _ = _sc_accumulate(hidden[:, :DH], rela_pad[:, :DH],
                                 alpha, sub, rel, obj)
    p0b, p1b, _ = _sc_accumulate(hidden[:, DH:], rela_pad[:, DH:],
                                 alpha, sub, rel, obj)
    return _mm_final(p0a, p1a, p0b, p1b, W_h, 1000)


# speedup vs baseline: 1.6873x; 1.0086x over previous
import functools

import jax
import jax.numpy as jnp
from jax import lax
from jax.experimental import pallas as pl
from jax.experimental.pallas import tpu as pltpu
from jax.experimental.pallas import tpu_sc as plsc

N_NODES = 10000
D = 256
DH = D // 2
VOCAB_PAD = 408
E_TOTAL = 160000
E_PAD = 163840
N_TILES = 16
NR = 632
NR_LAST = N_NODES - 15 * NR
MAGIC = 26547
CA = 32
R_ROUTE = 512
CAP = 576
CM = 64
EID_BITS = 18
EID_MASK = (1 << EID_BITS) - 1

_GATHER_DNUMS = lax.GatherDimensionNumbers(
    offset_dims=(), collapsed_slice_dims=(0,), start_index_map=(0,))


def _lane_shuffle(v, idx):
    return lax.gather(v, idx[:, None], _GATHER_DNUMS, (1,),
                      mode=lax.GatherScatterMode.PROMISE_IN_BOUNDS)


def _mm_bias(x, w, b, block_rows):
    R, d = x.shape
    a = w.shape[0]

    def body(x_ref, w_ref, b_ref, o_ref):
        o_ref[...] = lax.dot_general(
            x_ref[...], w_ref[...], (((1,), (1,)), ((), ())),
            preferred_element_type=jnp.float32) + b_ref[...]

    return pl.pallas_call(
        body,
        grid=(R // block_rows,),
        in_specs=[pl.BlockSpec((block_rows, d), lambda i: (i, 0)),
                  pl.BlockSpec((d, a), lambda i: (0, 0)),
                  pl.BlockSpec((1, a), lambda i: (0, 0))],
        out_specs=pl.BlockSpec((block_rows, a), lambda i: (i, 0)),
        out_shape=jax.ShapeDtypeStruct((R, a), jnp.float32),
    )(x, w, b)


def _mm_final(p0a, p1a, p0b, p1b, w, block_rows):
    R = p0a.shape[0]
    a = w.shape[0]

    def body(a0, a1, b0, b1, w_ref, o_ref):
        x = jnp.concatenate([a0[...] + a1[...], b0[...] + b1[...]], axis=1)
        o_ref[...] = lax.dot_general(
            x, w_ref[...], (((1,), (1,)), ((), ())),
            preferred_element_type=jnp.float32)

    half = pl.BlockSpec((block_rows, DH), lambda i: (i, 0))
    return pl.pallas_call(
        body,
        grid=(R // block_rows,),
        in_specs=[half, half, half, half,
                  pl.BlockSpec((D, a), lambda i: (0, 0))],
        out_specs=pl.BlockSpec((block_rows, a), lambda i: (i, 0)),
        out_shape=jax.ShapeDtypeStruct((R, a), jnp.float32),
    )(p0a, p1a, p0b, p1b, w)


def _sc_alpha(hid_attn, rel_attn, r2, q_rel, wal_ext, sub, rel, ridx):
    mesh = plsc.VectorSubcoreMesh(core_axis_name="c", subcore_axis_name="s")
    e_per_w = E_PAD // (2 * N_TILES)
    n_chunks = e_per_w // CA
    QC = 512

    @functools.partial(
        pl.kernel,
        mesh=mesh,
        out_type=jax.ShapeDtypeStruct((E_PAD,), jnp.float32),
        scratch_types=[
            pltpu.VMEM((D + 16,), jnp.float32),
            pltpu.VMEM((QC,), jnp.int32),
            pltpu.VMEM((e_per_w + CA,), jnp.int32),
            pltpu.VMEM((2, CA), jnp.int32),
            pltpu.VMEM((2, CA), jnp.int32),
            pltpu.VMEM((2, CA, D), jnp.float32),
            pltpu.VMEM((2, CA, D), jnp.float32),
            pltpu.VMEM((2, CA, D), jnp.float32),
            pltpu.VMEM((CA,), jnp.float32),
            pltpu.SemaphoreType.DMA,
            pltpu.SemaphoreType.DMA,
            pltpu.SemaphoreType.DMA,
            pltpu.SemaphoreType.DMA,
            pltpu.SemaphoreType.DMA,
            pltpu.SemaphoreType.DMA,
        ],
    )
    def body(ha_hbm, ra_hbm, r2_hbm, qrel_hbm, wal_hbm, sub_hbm, rel_hbm,
             ridx_hbm, alpha_hbm, wal_v, rq_v, qall_v, sub_v, rel_v,
             ha_v, ra_v, cq_v, al_v, *sems):
        c = lax.axis_index("c")
        s = lax.axis_index("s")
        w = s * 2 + c
        base0 = w * e_per_w
        pltpu.sync_copy(wal_hbm, wal_v)
        lanes = lax.iota(jnp.int32, 16)
        wal_b = wal_v[pl.ds(D, 16)]
        wal_regs = [wal_v[pl.ds(j * 16, 16)] for j in range(D // 16)]

        def qpass(t, carry):
            pltpu.sync_copy(ridx_hbm.at[pl.ds(base0 + t * QC, QC)], rq_v)
            pltpu.async_copy(qrel_hbm.at[rq_v],
                             qall_v.at[pl.ds(t * QC, QC)], sems[0]).wait()
            return carry

        lax.fori_loop(0, e_per_w // QC, qpass, 0)
        for g in range(CA // 16):
            qall_v[pl.ds(e_per_w + g * 16, 16)] = jnp.zeros((16,), jnp.int32)

        def issue(t, p):
            base = base0 + t * CA
            pltpu.sync_copy(sub_hbm.at[pl.ds(base, CA)], sub_v.at[p])
            pltpu.sync_copy(rel_hbm.at[pl.ds(base, CA)], rel_v.at[p])
            c1 = pltpu.async_copy(ha_hbm.at[sub_v.at[p]], ha_v.at[p],
                                  sems[3 * p])
            c2 = pltpu.async_copy(ra_hbm.at[rel_v.at[p]], ra_v.at[p],
                                  sems[3 * p + 1])
            c3 = pltpu.async_copy(r2_hbm.at[qall_v.at[pl.ds(t * CA, CA)]],
                                  cq_v.at[p], sems[3 * p + 2])
            return c1, c2, c3

        def compute(t, p):
            def group_body(g, carry2):
                avec = jnp.zeros((16,), jnp.float32)
                for e in range(16):
                    acc = jnp.zeros((16,), jnp.float32)
                    ei = g * 16 + e
                    for j in range(D // 16):
                        x = (ha_v[p, ei, pl.ds(j * 16, 16)]
                             + ra_v[p, ei, pl.ds(j * 16, 16)]
                             + cq_v[p, ei, pl.ds(j * 16, 16)])
                        acc = acc + jnp.maximum(x, 0.0) * wal_regs[j]
                    for sh in (8, 4, 2, 1):
                        acc = acc + _lane_shuffle(acc, lanes ^ sh)
                    av = 1.0 / (1.0 + jnp.exp(-(acc + wal_b)))
                    avec = jnp.where(lanes == e, av, avec)
                al_v[pl.ds(g * 16, 16)] = avec
                return carry2

            lax.fori_loop(0, CA // 16, group_body, 0)
            pltpu.sync_copy(al_v, alpha_hbm.at[pl.ds(base0 + t * CA, CA)])

        cps = issue(0, 0)

        def chunk_body(t2, carry):
            t = t2 * 2
            n1 = issue(t + 1, 1)
            for cp in cps:
                cp.wait()
            compute(t, 0)
            n0 = issue(t + 2, 0)
            for cp in n1:
                cp.wait()
            compute(t + 1, 1)
            return carry

        lax.fori_loop(0, n_chunks // 2, chunk_body, 0)
        for cp in cps:
            cp.wait()

    return body(hid_attn, rel_attn, r2, q_rel, wal_ext, sub, rel, ridx)


def _sc_accumulate(hid_h, rel_h, alpha, sub, rel, obj):
    mesh = plsc.VectorSubcoreMesh(core_axis_name="c", subcore_axis_name="s")
    e_per_w = E_PAD // (2 * N_TILES)
    n_rounds = e_per_w // R_ROUTE

    @functools.partial(
        pl.kernel,
        mesh=mesh,
        out_type=(jax.ShapeDtypeStruct((N_NODES, DH), jnp.float32),
                  jax.ShapeDtypeStruct((N_NODES, DH), jnp.float32),
                  jax.ShapeDtypeStruct((2 * N_TILES * 16,), jnp.int32)),
        scratch_types=[
            pltpu.VMEM((NR, DH), jnp.float32),
            pltpu.VMEM((N_TILES * CAP,), jnp.int32),
            pltpu.VMEM((CAP,), jnp.int32),
            pltpu.VMEM((R_ROUTE,), jnp.int32),
            pltpu.VMEM((CM,), jnp.int32),
            pltpu.VMEM((CM,), jnp.int32),
            pltpu.VMEM((CM,), jnp.int32),
            pltpu.VMEM((CM,), jnp.float32),
            pltpu.VMEM((32,), jnp.int32),
            pltpu.VMEM((16,), jnp.int32),
            pltpu.VMEM((CM, DH), jnp.float32),
            pltpu.VMEM((CM, DH), jnp.float32),
            pltpu.VMEM_SHARED((N_TILES * N_TILES * CAP,), jnp.int32),
            pltpu.SMEM((N_TILES,), jnp.int32),
            pltpu.SemaphoreType.DMA,
            pltpu.SemaphoreType.DMA,
            pltpu.SemaphoreType.DMA,
        ],
    )
    def body(hid_hbm, rel_hbm, al_hbm, sub_hbm, rele_hbm, obj_hbm,
             out0_hbm, out1_hbm, cnts_hbm, acc, lst, work, objs, eidc,
             subc, relc, alc, cntv, csrc, hs_v, hr_v, exch, cnt,
             sem1, sem2, sem3):
        c = lax.axis_index("c")
        s = lax.axis_index("s")
        lanes = lax.iota(jnp.int32, 16)
        zv = jnp.zeros((16,), jnp.float32)
        node_base = s * NR
        cidx = c * (N_TILES * 16) + lanes * 16 + s

        def zero_acc(r, carry):
            for j in range(DH // 16):
                acc[r, pl.ds(j * 16, 16)] = zv
            return carry

        lax.fori_loop(0, NR, zero_acc, 0)

        def round_body(r, carry):
            ebase = c * (E_PAD // 2) + s * e_per_w + r * R_ROUTE
            pltpu.sync_copy(obj_hbm.at[pl.ds(ebase, R_ROUTE)], objs)
            for d in range(N_TILES):
                cnt[d] = 0

            def route_body(g, carry2):
                ov = objs[pl.ds(g * 16, 16)]
                for e in range(16):
                    o = ov[e]
                    eid = ebase + g * 16 + e

                    @pl.when(o >= 0)
                    def _(o=o, eid=eid):
                        d = (o * MAGIC) >> 24
                        packed = eid | ((o - d * NR) << EID_BITS)
                        cn = cnt[d]
                        lst[pl.ds(d * CAP + cn, 16)] = jnp.full(
                            (16,), packed, jnp.int32)
                        cnt[d] = cn + 1
                return carry2

            lax.fori_loop(0, R_ROUTE // 16, route_body, 0)

            for d in range(N_TILES):
                cntv[pl.ds(d, 16)] = jnp.full((16,), cnt[d], jnp.int32)
            pltpu.sync_copy(cntv.at[pl.ds(0, 16)],
                            cnts_hbm.at[pl.ds((c * N_TILES + s) * 16, 16)])
            pltpu.sync_copy(lst, exch.at[pl.ds(s * (N_TILES * CAP),
                                               N_TILES * CAP)])
            plsc.subcore_barrier()
            pltpu.async_copy(cnts_hbm.at[cidx], csrc, sem1).wait()
            cv = csrc[pl.ds(0, 16)]

            def drain_body(src, carry2_outer):
                nv = _lane_shuffle(cv, (lanes + src) & 15)
                n_src = nv[0]

                def do_src():
                    pltpu.sync_copy(
                        exch.at[pl.ds(src * (N_TILES * CAP) + s * CAP,
                                      CAP)], work)
                    n_chunks = (n_src + CM - 1) >> 6

                    def chunk_body(k, carry2):
                        kbase = k * CM
                        vs = [work[pl.ds(kbase + q * 16, 16)]
                              for q in range(CM // 16)]
                        for q in range(CM // 16):
                            eidc[pl.ds(q * 16, 16)] = jnp.minimum(
                                vs[q] & EID_MASK, E_PAD - 1)
                        g1 = pltpu.async_copy(sub_hbm.at[eidc], subc, sem1)
                        g2 = pltpu.async_copy(rele_hbm.at[eidc], relc, sem2)
                        g4 = pltpu.async_copy(al_hbm.at[eidc], alc, sem3)
                        g1.wait()
                        g2.wait()
                        g4.wait()
                        g5 = pltpu.async_copy(hid_hbm.at[subc], hs_v, sem1)
                        g6 = pltpu.async_copy(rel_hbm.at[relc], hr_v, sem2)
                        g5.wait()
                        g6.wait()
                        for q in range(CM // 16):
                            rawv = vs[q]
                            av = alc[pl.ds(q * 16, 16)]
                            for e in range(16):
                                ei = q * 16 + e

                                @pl.when(kbase + ei < n_src)
                                def _(ei=ei,
                                      row=rawv[e] >> EID_BITS,
                                      asc=av[e]):
                                    afull = jnp.full((16,), asc,
                                                     jnp.float32)
                                    for j in range(DH // 16):
                                        m = (hs_v[ei, pl.ds(j * 16, 16)]
                                             + hr_v[ei, pl.ds(j * 16, 16)])
                                        plsc.addupdate(
                                            acc.at[row, pl.ds(j * 16, 16)],
                                            afull * m)

                        return carry2

                    lax.fori_loop(0, n_chunks, chunk_body, 0)

                pl.when(n_src > 0)(do_src)
                return carry2_outer

            lax.fori_loop(0, N_TILES, drain_body, 0)
            plsc.subcore_barrier()
            return carry

        lax.fori_loop(0, n_rounds, round_body, 0)

        def write_to(out_hbm):
            pl.when(s < N_TILES - 1)(lambda: pltpu.sync_copy(
                acc.at[pl.ds(0, NR)], out_hbm.at[pl.ds(node_base, NR)]))
            pl.when(s == N_TILES - 1)(lambda: pltpu.sync_copy(
                acc.at[pl.ds(0, NR_LAST)],
                out_hbm.at[pl.ds(node_base, NR_LAST)]))

        pl.when(c == 0)(lambda: write_to(out0_hbm))
        pl.when(c == 1)(lambda: write_to(out1_hbm))

    return body(hid_h, rel_h, alpha, sub, rel, obj)


def kernel(q_sub, q_rel, hidden, edges, n_node, old_nodes_new_idx, rela_embed,
           Ws, Wr, Wqr_w, Wqr_b, walpha_w, walpha_b, W_h):
    pad = E_PAD - E_TOTAL
    sub = jnp.pad(edges[:, 4], (0, pad + CA))
    rel = jnp.pad(edges[:, 2], (0, pad + CA))
    obj = jnp.pad(edges[:, 5], (0, pad), constant_values=-1)
    ridx = jnp.pad(edges[:, 0], (0, pad))
    rela_pad = jnp.pad(rela_embed, ((0, VOCAB_PAD - rela_embed.shape[0]),
                                    (0, 0)))
    zb = jnp.zeros((1, D), jnp.float32)
    hid_attn = _mm_bias(hidden, Ws, zb, 1000)
    rel_attn = _mm_bias(rela_pad, Wr, zb, VOCAB_PAD)
    r2 = _mm_bias(rela_pad, Wqr_w, Wqr_b.reshape(1, -1), VOCAB_PAD)
    wal_ext = jnp.concatenate(
        [walpha_w[0], jnp.full((16,), walpha_b[0], jnp.float32)])

    alpha = _sc_alpha(hid_attn, rel_attn, r2, q_rel, wal_ext, sub, rel, ridx)

    p0a, p1a, _ = _sc_accumulate(hidden[:, :DH], rela_pad[:, :DH],
                                 alpha, sub, rel, obj)
    p0b, p1b, _ = _sc_accumulate(hidden[:, DH:], rela_pad[:, DH:],
                                 alpha, sub, rel, obj)
    return _mm_final(p0a, p1a, p0b, p1b, W_h, 1000)
